# Initial kernel scaffold; baseline (speedup 1.0000x reference)
import jax
import jax.numpy as jnp
from jax.experimental import pallas as pl


def kernel(base, intervention_positions, batch_rotation, batch_weights):
    raise NotImplementedError("write your pallas kernel here")


# speedup vs baseline: 1.5647x; 1.5647x over previous
import functools

import jax
import jax.numpy as jnp
from jax import lax
from jax.experimental import pallas as pl
from jax.experimental.pallas import tpu as pltpu
from jax.experimental.pallas import tpu_sc as plsc

B, S, H, P, LR = 4, 4096, 2048, 128, 8


def _sc_gather(base_flat, pos_flat):
    info = plsc.get_sparse_core_info()
    nc, ns = info.num_cores, info.num_subcores
    nw = nc * ns
    rows_total = B * P
    b_per_w = rows_total // nw

    mesh = plsc.VectorSubcoreMesh(core_axis_name="c", subcore_axis_name="s")

    @functools.partial(
        pl.kernel,
        out_type=jax.ShapeDtypeStruct((rows_total, H), jnp.float32),
        mesh=mesh,
        scratch_types=[
            pltpu.VMEM((b_per_w,), jnp.int32),
            pltpu.VMEM((b_per_w, H), jnp.float32),
            pltpu.SemaphoreType.DMA,
        ],
    )
    def k(base_hbm, idx_hbm, out_hbm, idx_v, rows_v, sem):
        wid = lax.axis_index("s") * nc + lax.axis_index("c")
        row0 = wid * b_per_w
        pltpu.sync_copy(idx_hbm.at[pl.ds(row0, b_per_w)], idx_v)
        batch = row0 // P
        idx_v[...] = idx_v[...] + batch * S
        pltpu.async_copy(base_hbm.at[idx_v], rows_v, sem).wait()
        pltpu.sync_copy(rows_v, out_hbm.at[pl.ds(row0, b_per_w)])

    return k(base_flat, pos_flat)


def _tc_mixed(gathered, rotation, weights):
    def body(g_ref, r_ref, w_ref, o_ref):
        g = g_ref[0]
        rot = r_ref[0]
        tmp = (jnp.dot(g, w_ref[0], preferred_element_type=jnp.float32)
               - jnp.dot(g, rot, preferred_element_type=jnp.float32))
        o_ref[0] = lax.dot_general(
            tmp, rot, (((1,), (1,)), ((), ())),
            preferred_element_type=jnp.float32)

    return pl.pallas_call(
        body,
        grid=(B,),
        in_specs=[
            pl.BlockSpec((1, P, H), lambda b: (b, 0, 0)),
            pl.BlockSpec((1, H, LR), lambda b: (b, 0, 0)),
            pl.BlockSpec((1, H, LR), lambda b: (b, 0, 0)),
        ],
        out_specs=pl.BlockSpec((1, P, H), lambda b: (b, 0, 0)),
        out_shape=jax.ShapeDtypeStruct((B, P, H), jnp.float32),
    )(gathered, rotation, weights)


_BLK = 1024


def _tc_copy_substitute(base, mixed, pos):
    def body(pos_ref, base_ref, mix_ref, o_ref):
        b = pl.program_id(0)
        s = pl.program_id(1)
        o_ref[...] = base_ref[...]
        start = s * _BLK

        def step(p, carry):
            off = pos_ref[b, p] - start

            @pl.when((off >= 0) & (off < _BLK))
            def _():
                o_ref[0, pl.ds(off, 1), :] = mix_ref[0, pl.ds(p, 1), :]

            return carry

        lax.fori_loop(0, P, step, 0)

    return pl.pallas_call(
        body,
        grid=(B, S // _BLK),
        in_specs=[
            pl.BlockSpec(memory_space=pltpu.SMEM),
            pl.BlockSpec((1, _BLK, H), lambda b, s: (b, s, 0)),
            pl.BlockSpec((1, P, H), lambda b, s: (b, 0, 0)),
        ],
        out_specs=pl.BlockSpec((1, _BLK, H), lambda b, s: (b, s, 0)),
        out_shape=jax.ShapeDtypeStruct((B, S, H), jnp.float32),
    )(pos, base, mixed)


def kernel(base, intervention_positions, batch_rotation, batch_weights):
    pos = intervention_positions.astype(jnp.int32)
    gathered = _sc_gather(base.reshape(B * S, H), pos.reshape(B * P))
    mixed = _tc_mixed(
        gathered.reshape(B, P, H),
        batch_rotation.reshape(B, H, LR),
        batch_weights.reshape(B, H, LR),
    )
    return _tc_copy_substitute(base, mixed, pos)


# speedup vs baseline: 1.6094x; 1.0286x over previous
import functools

import jax
import jax.numpy as jnp
from jax import lax
from jax.experimental import pallas as pl
from jax.experimental.pallas import tpu as pltpu
from jax.experimental.pallas import tpu_sc as plsc

B, S, H, P, LR = 4, 4096, 2048, 128, 8


def _sc_gather(base_flat, pos_flat):
    info = plsc.get_sparse_core_info()
    nc, ns = info.num_cores, info.num_subcores
    nw = nc * ns
    rows_total = B * P
    b_per_w = rows_total // nw

    mesh = plsc.VectorSubcoreMesh(core_axis_name="c", subcore_axis_name="s")

    @functools.partial(
        pl.kernel,
        out_type=jax.ShapeDtypeStruct((rows_total, H), jnp.float32),
        mesh=mesh,
        scratch_types=[
            pltpu.VMEM((b_per_w,), jnp.int32),
            pltpu.VMEM((b_per_w, H), jnp.float32),
            pltpu.SemaphoreType.DMA,
        ],
    )
    def k(base_hbm, idx_hbm, out_hbm, idx_v, rows_v, sem):
        wid = lax.axis_index("s") * nc + lax.axis_index("c")
        row0 = wid * b_per_w
        pltpu.sync_copy(idx_hbm.at[pl.ds(row0, b_per_w)], idx_v)
        batch = row0 // P
        idx_v[...] = idx_v[...] + batch * S
        pltpu.async_copy(base_hbm.at[idx_v], rows_v, sem).wait()
        pltpu.sync_copy(rows_v, out_hbm.at[pl.ds(row0, b_per_w)])

    return k(base_flat, pos_flat)


def _tc_mix_scatter(base, gathered, rotation, weights, pos):
    def body(pos_ref, base_ref, g_ref, r_ref, w_ref, out_ref, mix_v, sem):
        del base_ref
        b = pl.program_id(0)
        g = g_ref[0]
        rot = r_ref[0]
        tmp = (jnp.dot(g, w_ref[0], preferred_element_type=jnp.float32)
               - jnp.dot(g, rot, preferred_element_type=jnp.float32))
        mix_v[...] = lax.dot_general(
            tmp, rot, (((1,), (1,)), ((), ())),
            preferred_element_type=jnp.float32)

        def issue(p, carry):
            pos_p = pos_ref[b, p]
            pltpu.make_async_copy(
                mix_v.at[pl.ds(p, 1), :],
                out_ref.at[b, pl.ds(pos_p, 1), :],
                sem,
            ).start()
            return carry

        lax.fori_loop(0, P, issue, 0)

        def drain(p, carry):
            pltpu.make_async_copy(
                mix_v.at[pl.ds(0, 1), :],
                out_ref.at[b, pl.ds(0, 1), :],
                sem,
            ).wait()
            return carry

        lax.fori_loop(0, P, drain, 0)

    return pl.pallas_call(
        body,
        grid=(B,),
        in_specs=[
            pl.BlockSpec(memory_space=pltpu.SMEM),
            pl.BlockSpec(memory_space=pltpu.MemorySpace.HBM),
            pl.BlockSpec((1, P, H), lambda b: (b, 0, 0)),
            pl.BlockSpec((1, H, LR), lambda b: (b, 0, 0)),
            pl.BlockSpec((1, H, LR), lambda b: (b, 0, 0)),
        ],
        out_specs=pl.BlockSpec(memory_space=pltpu.MemorySpace.HBM),
        out_shape=jax.ShapeDtypeStruct((B, S, H), jnp.float32),
        scratch_shapes=[
            pltpu.VMEM((P, H), jnp.float32),
            pltpu.SemaphoreType.DMA,
        ],
        input_output_aliases={1: 0},
    )(pos, base, gathered, rotation, weights)


def kernel(base, intervention_positions, batch_rotation, batch_weights):
    pos = intervention_positions.astype(jnp.int32)
    gathered = _sc_gather(base.reshape(B * S, H), pos.reshape(B * P))
    return _tc_mix_scatter(
        base,
        gathered.reshape(B, P, H),
        batch_rotation.reshape(B, H, LR),
        batch_weights.reshape(B, H, LR),
        pos,
    )
